# baseline (speedup 1.0000x reference)
import jax
from jax import numpy as jnp
from jax import lax
from jax.experimental import pallas as pl
from jax.experimental.pallas import tpu as pltpu
from jax.experimental.pallas import tpu_sc as plsc

_B = 16384
_EMB = 64
_L = 16
_NW = 32
_BW = _B // _NW
_CH = 16
_NCH = _BW // _CH


def _gat_body(users_ref, items_ref, utab, itab, wuref, wiref, bref, out_ref,
              ubuf, ibuf, urow, irow, usel, isel, ug0, ug1, ig0, ig1, outb,
              wubuf, wibuf, bbuf, sem0, sem1):
    wid = lax.axis_index('s') * 2 + lax.axis_index('c')
    base = wid * _BW

    pltpu.sync_copy(users_ref.at[pl.ds(base, _BW)], ubuf)
    pltpu.sync_copy(items_ref.at[pl.ds(base, _BW)], ibuf)
    pltpu.sync_copy(wuref, wubuf)
    pltpu.sync_copy(wiref, wibuf)
    pltpu.sync_copy(bref, bbuf)

    def rows_body(t, carry):
        uv = ubuf[pl.ds(t * _L, _L)]
        iv = ibuf[pl.ds(t * _L, _L)]
        urow[pl.ds(t * _L, _L)] = uv >> 1
        irow[pl.ds(t * _L, _L)] = iv >> 1
        usel[pl.ds(t * _L, _L)] = (uv & 1) * _EMB
        isel[pl.ds(t * _L, _L)] = (iv & 1) * _EMB
        return carry

    lax.fori_loop(0, _BW // _L, rows_body, 0)

    bv = bbuf[...]
    iota = lax.iota(jnp.int32, _L)
    ugs = [ug0, ug1]
    igs = [ig0, ig1]
    sems = [sem0, sem1]

    def fire(c):
        s = sems[c % 2]
        du = pltpu.async_copy(
            utab.at[urow.at[pl.ds(c * _CH, _CH)]],
            ugs[c % 2], s)
        di = pltpu.async_copy(
            itab.at[irow.at[pl.ds(c * _CH, _CH)]],
            igs[c % 2], s)
        return (du, di)

    pending = fire(0)
    for c in range(_NCH):
        nxt = fire(c + 1) if c + 1 < _NCH else None
        du, di = pending
        du.wait()
        di.wait()
        ug = ugs[c % 2]
        ig = igs[c % 2]
        us = usel[pl.ds(c * _CH, _CH)]
        sl = isel[pl.ds(c * _CH, _CH)]

        def kstep(k, acc):
            k16 = jnp.full((_L,), 0, jnp.int32) + k
            uval = plsc.load_gather(ug, [iota, us + k16])
            ival = plsc.load_gather(ig, [iota, sl + k16])
            wk = plsc.load_gather(wubuf, [k16])
            vk = plsc.load_gather(wibuf, [k16])
            return acc + uval * wk + ival * vk

        acc = lax.fori_loop(0, _EMB, kstep, bv)
        outb[pl.ds(c * _CH, _CH)] = acc
        pending = nxt

    pltpu.sync_copy(outb, out_ref.at[pl.ds(base, _BW)])


def kernel(users, items, user_table, item_table, W, b):
    users1d = users.astype(jnp.int32)
    items1d = items.astype(jnp.int32)
    wflat = W.reshape(2 * _EMB)
    wu = wflat[:_EMB]
    wi = wflat[_EMB:]
    b16 = jnp.broadcast_to(b, (_L,))

    utab2 = user_table.reshape(user_table.shape[0] // 2, 2 * _EMB)
    itab2 = item_table.reshape(item_table.shape[0] // 2, 2 * _EMB)

    mesh = plsc.VectorSubcoreMesh(core_axis_name='c', subcore_axis_name='s')
    f = pl.kernel(
        _gat_body,
        out_type=jax.ShapeDtypeStruct((_B,), jnp.float32),
        mesh=mesh,
        compiler_params=pltpu.CompilerParams(
            needs_layout_passes=False),
        scratch_types=[
            pltpu.VMEM((_BW,), jnp.int32),
            pltpu.VMEM((_BW,), jnp.int32),
            pltpu.VMEM((_BW,), jnp.int32),
            pltpu.VMEM((_BW,), jnp.int32),
            pltpu.VMEM((_BW,), jnp.int32),
            pltpu.VMEM((_BW,), jnp.int32),
            pltpu.VMEM((_CH, 2 * _EMB), jnp.float32),
            pltpu.VMEM((_CH, 2 * _EMB), jnp.float32),
            pltpu.VMEM((_CH, 2 * _EMB), jnp.float32),
            pltpu.VMEM((_CH, 2 * _EMB), jnp.float32),
            pltpu.VMEM((_BW,), jnp.float32),
            pltpu.VMEM((_EMB,), jnp.float32),
            pltpu.VMEM((_EMB,), jnp.float32),
            pltpu.VMEM((_L,), jnp.float32),
            pltpu.SemaphoreType.DMA,
            pltpu.SemaphoreType.DMA,
        ],
    )
    out = f(users1d, items1d, utab2, itab2, wu, wi, b16)
    return out.reshape(_B, 1)


# speedup vs baseline: 1.1817x; 1.1817x over previous
import jax
from jax import numpy as jnp
from jax import lax
from jax.experimental import pallas as pl
from jax.experimental.pallas import tpu as pltpu
from jax.experimental.pallas import tpu_sc as plsc

_B = 16384
_EMB = 64
_L = 16
_NW = 32
_BW = _B // _NW
_CH = 16
_NCH = _BW // _CH
_RB = 16384
_D = 4


def _make_proj_body(nrows, nb):
    def _proj_body(tab_ref, w_ref, o_ref, b0, b1, b2, b3, s0, s1, s2, s3):
        bufs = [b0, b1, b2, b3]
        sems = [s0, s1, s2, s3]
        w = w_ref[...].reshape(1, 1, _EMB)

        def issue(n):
            rows = min(_RB, nrows - n * _RB)
            cp = pltpu.make_async_copy(
                tab_ref.at[pl.ds(n * _RB, rows)],
                bufs[n % _D].at[pl.ds(0, rows)],
                sems[n % _D])
            cp.start()
            return cp

        cps = {}
        for n in range(min(_D, nb)):
            cps[n] = issue(n)
        for n in range(nb):
            cps[n].wait()
            x = bufs[n % _D][...]
            o_ref[pl.ds(n * (_RB // 128), _RB // 128)] = jnp.sum(
                x.reshape(_RB // 128, 128, _EMB) * w, axis=-1)
            if n + _D < nb:
                cps[n + _D] = issue(n + _D)

    return _proj_body


def _project(table, w2d):
    nrows = table.shape[0]
    nb = -(-nrows // _RB)
    return pl.pallas_call(
        _make_proj_body(nrows, nb),
        out_shape=jax.ShapeDtypeStruct((nb * (_RB // 128), 128),
                                       jnp.float32),
        in_specs=[pl.BlockSpec(memory_space=pl.ANY),
                  pl.BlockSpec(memory_space=pltpu.MemorySpace.VMEM)],
        out_specs=pl.BlockSpec(memory_space=pltpu.MemorySpace.VMEM),
        scratch_shapes=[pltpu.VMEM((_RB, _EMB), jnp.float32)] * _D
        + [pltpu.SemaphoreType.DMA] * _D,
    )(table, w2d)


def _gat_body(users_ref, items_ref, uproj, iproj, bref, out_ref,
              ubuf, ibuf, urow, irow, ug0, ug1, ig0, ig1, outb,
              bbuf, sem0, sem1):
    wid = lax.axis_index('s') * 2 + lax.axis_index('c')
    base = wid * _BW

    pltpu.sync_copy(users_ref.at[pl.ds(base, _BW)], ubuf)
    pltpu.sync_copy(items_ref.at[pl.ds(base, _BW)], ibuf)
    pltpu.sync_copy(bref, bbuf)

    def rows_body(t, carry):
        urow[pl.ds(t * _L, _L)] = ubuf[pl.ds(t * _L, _L)] >> 7
        irow[pl.ds(t * _L, _L)] = ibuf[pl.ds(t * _L, _L)] >> 7
        return carry

    lax.fori_loop(0, _BW // _L, rows_body, 0)

    bv = bbuf[...]
    iota = lax.iota(jnp.int32, _L)
    ugs = [ug0, ug1]
    igs = [ig0, ig1]
    sems = [sem0, sem1]

    def fire(c):
        s = sems[c % 2]
        du = pltpu.async_copy(
            uproj.at[urow.at[pl.ds(c * _CH, _CH)]],
            ugs[c % 2], s)
        di = pltpu.async_copy(
            iproj.at[irow.at[pl.ds(c * _CH, _CH)]],
            igs[c % 2], s)
        return (du, di)

    pending = fire(0)
    for c in range(_NCH):
        nxt = fire(c + 1) if c + 1 < _NCH else None
        du, di = pending
        du.wait()
        di.wait()
        ug = ugs[c % 2]
        ig = igs[c % 2]
        uv = ubuf[pl.ds(c * _CH, _CH)]
        iv = ibuf[pl.ds(c * _CH, _CH)]
        uval = plsc.load_gather(ug, [iota, uv & 127])
        ival = plsc.load_gather(ig, [iota, iv & 127])
        outb[pl.ds(c * _CH, _CH)] = uval + ival + bv
        pending = nxt

    pltpu.sync_copy(outb, out_ref.at[pl.ds(base, _BW)])


def kernel(users, items, user_table, item_table, W, b):
    users1d = users.astype(jnp.int32)
    items1d = items.astype(jnp.int32)
    wflat = W.reshape(2 * _EMB)
    wu = wflat[:_EMB].reshape(1, _EMB)
    wi = wflat[_EMB:].reshape(1, _EMB)
    b16 = jnp.broadcast_to(b, (_L,))

    uproj = _project(user_table, wu)
    iproj = _project(item_table, wi)

    mesh = plsc.VectorSubcoreMesh(core_axis_name='c', subcore_axis_name='s')
    f = pl.kernel(
        _gat_body,
        out_type=jax.ShapeDtypeStruct((_B,), jnp.float32),
        mesh=mesh,
        compiler_params=pltpu.CompilerParams(needs_layout_passes=False),
        scratch_types=[
            pltpu.VMEM((_BW,), jnp.int32),
            pltpu.VMEM((_BW,), jnp.int32),
            pltpu.VMEM((_BW,), jnp.int32),
            pltpu.VMEM((_BW,), jnp.int32),
            pltpu.VMEM((_CH, 128), jnp.float32),
            pltpu.VMEM((_CH, 128), jnp.float32),
            pltpu.VMEM((_CH, 128), jnp.float32),
            pltpu.VMEM((_CH, 128), jnp.float32),
            pltpu.VMEM((_BW,), jnp.float32),
            pltpu.VMEM((_L,), jnp.float32),
            pltpu.SemaphoreType.DMA,
            pltpu.SemaphoreType.DMA,
        ],
    )
    out = f(users1d, items1d, uproj, iproj, b16)
    return out.reshape(_B, 1)
